# baseline (speedup 1.0000x reference)
import functools

import jax
import jax.numpy as jnp
from jax import lax
from jax.experimental import pallas as pl
from jax.experimental.pallas import tpu as pltpu
from jax.experimental.pallas import tpu_sc as plsc

N = 160000
D = 256
DIM = 10000
NC = 2
NS = 16
L = 16
SPAD = 10240
SEG_SC = SPAD // NC
SEG_W = SPAD // (NC * NS)
EPT = N // NS
VPT = EPT // L
NT = 2 * NS
ROWS_CHUNK = 64
NCHUNK = SEG_W // ROWS_CHUNK
TAIL = DIM % ROWS_CHUNK
MARKER = 2147483647


def _body(idx_hbm, t_hbm, msg_hbm, out_hbm,
          idx_v, t_v, segmax_a, argmax_a, segmax_b, argmax_b,
          amax_rows, rows_v, rows2_v, sem, sem2,
          shared_f, shared_i):
  merge_i = idx_v
  merge_f = t_v
  c = lax.axis_index("c")
  s = lax.axis_index("s")
  sc_base = c * SEG_SC

  stage_idx = pltpu.async_copy(idx_hbm.at[pl.ds(s * EPT, EPT)],
                               idx_v.at[pl.ds(0, EPT)], sem)
  stage_t = pltpu.async_copy(t_hbm.at[pl.ds(s * EPT, EPT)],
                             t_v.at[pl.ds(0, EPT)], sem2)

  neg_inf = jnp.full((L,), -jnp.inf, jnp.float32)

  def init_body(j, carry):
    segmax_a[pl.ds(sc_base + j * L, L)] = neg_inf
    segmax_b[pl.ds(sc_base + j * L, L)] = neg_inf
    return carry

  lax.fori_loop(0, SEG_SC // L, init_body, 0)
  stage_idx.wait()
  stage_t.wait()

  lanes = lax.iota(jnp.int32, L)
  elem_base = s * EPT

  def stage1(vi):
    iv = idx_v[pl.ds(vi * L, L)]
    tv = t_v[pl.ds(vi * L, L)]
    stv, sval = plsc.sort_key_val(tv, iv * L + lanes)
    siv = lax.shift_right_logical(sval, 4)
    spos = elem_base + vi * L + (sval & (L - 1))
    return stv, siv, spos

  def stage2(st, seg_ref, arg_ref):
    stv, siv, spos = st
    cm = plsc.load_gather(seg_ref, [siv])
    wm = stv >= cm
    plsc.store_scatter(seg_ref, [siv], stv, mask=wm)
    plsc.store_scatter(arg_ref, [siv], spos, mask=wm)

  def pair_body(i, carry):
    sa = stage1(2 * i)
    sb = stage1(2 * i + 1)
    stage2(sa, segmax_a, argmax_a)
    stage2(sb, segmax_b, argmax_b)
    return carry

  lax.fori_loop(0, VPT // 2, pair_body, 0)
  if VPT % 2:
    stage2(stage1(VPT - 1), segmax_a, argmax_a)

  plsc.subcore_barrier()

  copies = []
  for r in range(NS):
    src = sc_base + r * SEG_W
    dst = (r * NT + 2 * s) * SEG_W
    copies.append(pltpu.async_copy(
        segmax_a.at[pl.ds(src, SEG_W)],
        shared_f.at[pl.ds(dst, SEG_W)], sem))
    copies.append(pltpu.async_copy(
        argmax_a.at[pl.ds(src, SEG_W)],
        shared_i.at[pl.ds(dst, SEG_W)], sem))
    copies.append(pltpu.async_copy(
        segmax_b.at[pl.ds(src, SEG_W)],
        shared_f.at[pl.ds(dst + SEG_W, SEG_W)], sem2))
    copies.append(pltpu.async_copy(
        argmax_b.at[pl.ds(src, SEG_W)],
        shared_i.at[pl.ds(dst + SEG_W, SEG_W)], sem2))
  for cp_ in copies:
    cp_.wait()
  plsc.subcore_barrier()

  pltpu.sync_copy(shared_f.at[pl.ds(s * NT * SEG_W, NT * SEG_W)], merge_f)
  pltpu.sync_copy(shared_i.at[pl.ds(s * NT * SEG_W, NT * SEG_W)], merge_i)

  out_base = sc_base + s * SEG_W

  def merge_body(j, carry):
    am = merge_f[pl.ds(j * L, L)]
    ap = merge_i[pl.ds(j * L, L)]
    for r in range(1, NT):
      m = merge_f[pl.ds(r * SEG_W + j * L, L)]
      p = merge_i[pl.ds(r * SEG_W + j * L, L)]
      b = m > am
      am = jnp.where(b, m, am)
      ap = jnp.where(b, p, ap)
    gseg = out_base + j * L + lanes
    fill = jnp.where(gseg >= DIM, gseg, jnp.int32(DIM - 1))
    ap = jnp.where(am == -jnp.inf, fill, ap)
    amax_rows[pl.ds(j * L, L)] = ap
    return carry

  lax.fori_loop(0, SEG_W // L, merge_body, 0)

  def chunk_gather(k, buf, gsem):
    return pltpu.async_copy(
        msg_hbm.at[amax_rows.at[pl.ds(k * ROWS_CHUNK, ROWS_CHUNK)]],
        buf, gsem)

  def chunk_write(k, buf):
    chunk_base = out_base + k * ROWS_CHUNK

    @pl.when(chunk_base + ROWS_CHUNK <= DIM)
    def _full_write():
      pltpu.sync_copy(buf, out_hbm.at[pl.ds(chunk_base, ROWS_CHUNK)])

    @pl.when(chunk_base == DIM - TAIL)
    def _tail_write():
      pltpu.sync_copy(buf.at[pl.ds(0, TAIL)],
                      out_hbm.at[pl.ds(DIM - TAIL, TAIL)])

  plsc.subcore_barrier()

  bufs = (rows_v, rows2_v)
  gsems = (sem, sem2)
  g = chunk_gather(0, bufs[0], gsems[0])
  for k in range(NCHUNK):
    g.wait()
    if k + 1 < NCHUNK:
      g = chunk_gather(k + 1, bufs[(k + 1) % 2], gsems[(k + 1) % 2])
    chunk_write(k, bufs[k % 2])


@jax.jit
def _aggregate(msg, index, t):
  mesh = plsc.VectorSubcoreMesh(core_axis_name="c", subcore_axis_name="s")
  run = functools.partial(
      pl.kernel,
      out_type=jax.ShapeDtypeStruct((DIM, D), jnp.float32),
      mesh=mesh,
      scratch_types=[
          pltpu.VMEM((NT * SEG_W,), jnp.int32),
          pltpu.VMEM((NT * SEG_W,), jnp.float32),
          pltpu.VMEM((SPAD,), jnp.float32),
          pltpu.VMEM((SPAD,), jnp.int32),
          pltpu.VMEM((SPAD,), jnp.float32),
          pltpu.VMEM((SPAD,), jnp.int32),
          pltpu.VMEM((SEG_W,), jnp.int32),
          pltpu.VMEM((ROWS_CHUNK, D), jnp.float32),
          pltpu.VMEM((ROWS_CHUNK, D), jnp.float32),
          pltpu.SemaphoreType.DMA,
          pltpu.SemaphoreType.DMA,
          pltpu.VMEM_SHARED((NS * NT * SEG_W,), jnp.float32),
          pltpu.VMEM_SHARED((NS * NT * SEG_W,), jnp.int32),
      ],
      compiler_params=pltpu.CompilerParams(needs_layout_passes=False),
  )(_body)
  return run(index, t, msg)


def kernel(msg, index, t, dim_size):
  del dim_size
  return _aggregate(msg, index, t)


# speedup vs baseline: 1.0015x; 1.0015x over previous
import functools

import jax
import jax.numpy as jnp
from jax import lax
from jax.experimental import pallas as pl
from jax.experimental.pallas import tpu as pltpu
from jax.experimental.pallas import tpu_sc as plsc

N = 160000
D = 256
DIM = 10000
NC = 2
NS = 16
L = 16
SPAD = 10240
SEG_SC = SPAD // NC
SEG_W = SPAD // (NC * NS)
EPT = N // NS
VPT = EPT // L
NT = 2 * NS
ROWS_CHUNK = 64
NCHUNK = SEG_W // ROWS_CHUNK
TAIL = DIM % ROWS_CHUNK


def _body(idx_hbm, t_hbm, msg_hbm, out_hbm,
          idx_v, t_v, segmax_a, argmax_a, segmax_b, argmax_b,
          amax_rows, rows_v, rows2_v, sem, sem2,
          shared_f, shared_i):
  merge_i = idx_v
  merge_f = t_v
  c = lax.axis_index("c")
  s = lax.axis_index("s")
  sc_base = c * SEG_SC

  stage_idx = pltpu.async_copy(idx_hbm.at[pl.ds(s * EPT, EPT)],
                               idx_v.at[pl.ds(0, EPT)], sem)
  stage_t = pltpu.async_copy(t_hbm.at[pl.ds(s * EPT, EPT)],
                             t_v.at[pl.ds(0, EPT)], sem2)

  neg_inf = jnp.full((L,), -jnp.inf, jnp.float32)

  def init_body(j, carry):
    segmax_a[pl.ds(sc_base + j * L, L)] = neg_inf
    segmax_b[pl.ds(sc_base + j * L, L)] = neg_inf
    return carry

  lax.fori_loop(0, SEG_SC // L, init_body, 0)
  stage_idx.wait()
  stage_t.wait()

  lanes = lax.iota(jnp.int32, L)
  elem_base = s * EPT

  def stage1(vi):
    iv = idx_v[pl.ds(vi * L, L)]
    tv = t_v[pl.ds(vi * L, L)]
    stv, sval = plsc.sort_key_val(tv, iv * L + lanes)
    siv = lax.shift_right_logical(sval, 4)
    spos = elem_base + vi * L + (sval & (L - 1))
    return stv, siv, spos

  def stage2(st, seg_ref, arg_ref):
    stv, siv, spos = st
    cm = plsc.load_gather(seg_ref, [siv])
    wm = stv >= cm
    plsc.store_scatter(seg_ref, [siv], stv, mask=wm)
    plsc.store_scatter(arg_ref, [siv], spos, mask=wm)

  def pair_body(i, carry):
    sa = stage1(2 * i)
    sb = stage1(2 * i + 1)
    stage2(sa, segmax_a, argmax_a)
    stage2(sb, segmax_b, argmax_b)
    return carry

  lax.fori_loop(0, VPT // 2, pair_body, 0)
  if VPT % 2:
    stage2(stage1(VPT - 1), segmax_a, argmax_a)

  plsc.subcore_barrier()

  copies = []
  for r in range(NS):
    src = sc_base + r * SEG_W
    dst = (r * NT + 2 * s) * SEG_W
    copies.append(pltpu.async_copy(
        segmax_a.at[pl.ds(src, SEG_W)],
        shared_f.at[pl.ds(dst, SEG_W)], sem))
    copies.append(pltpu.async_copy(
        argmax_a.at[pl.ds(src, SEG_W)],
        shared_i.at[pl.ds(dst, SEG_W)], sem))
    copies.append(pltpu.async_copy(
        segmax_b.at[pl.ds(src, SEG_W)],
        shared_f.at[pl.ds(dst + SEG_W, SEG_W)], sem2))
    copies.append(pltpu.async_copy(
        argmax_b.at[pl.ds(src, SEG_W)],
        shared_i.at[pl.ds(dst + SEG_W, SEG_W)], sem2))
  for cp_ in copies:
    cp_.wait()
  plsc.subcore_barrier()

  pltpu.sync_copy(shared_f.at[pl.ds(s * NT * SEG_W, NT * SEG_W)], merge_f)
  pltpu.sync_copy(shared_i.at[pl.ds(s * NT * SEG_W, NT * SEG_W)], merge_i)

  out_base = sc_base + s * SEG_W

  def merge_body(j, carry):
    am = merge_f[pl.ds(j * L, L)]
    ap = merge_i[pl.ds(j * L, L)]
    for r in range(1, NT):
      m = merge_f[pl.ds(r * SEG_W + j * L, L)]
      p = merge_i[pl.ds(r * SEG_W + j * L, L)]
      b = m > am
      am = jnp.where(b, m, am)
      ap = jnp.where(b, p, ap)
    gseg = out_base + j * L + lanes
    fill = jnp.where(gseg >= DIM, gseg, jnp.int32(DIM - 1))
    ap = jnp.where(am == -jnp.inf, fill, ap)
    amax_rows[pl.ds(j * L, L)] = ap
    return carry

  lax.fori_loop(0, SEG_W // L, merge_body, 0)

  def chunk_gather(k, buf, gsem):
    return pltpu.async_copy(
        msg_hbm.at[amax_rows.at[pl.ds(k * ROWS_CHUNK, ROWS_CHUNK)]],
        buf, gsem)

  def chunk_write(k, buf):
    chunk_base = out_base + k * ROWS_CHUNK

    @pl.when(chunk_base + ROWS_CHUNK <= DIM)
    def _full_write():
      pltpu.sync_copy(buf, out_hbm.at[pl.ds(chunk_base, ROWS_CHUNK)])

    @pl.when(chunk_base == DIM - TAIL)
    def _tail_write():
      pltpu.sync_copy(buf.at[pl.ds(0, TAIL)],
                      out_hbm.at[pl.ds(DIM - TAIL, TAIL)])

  plsc.subcore_barrier()

  bufs = (rows_v, rows2_v)
  gsems = (sem, sem2)
  g = chunk_gather(0, bufs[0], gsems[0])
  for k in range(NCHUNK):
    g.wait()
    if k + 1 < NCHUNK:
      g = chunk_gather(k + 1, bufs[(k + 1) % 2], gsems[(k + 1) % 2])
    chunk_write(k, bufs[k % 2])


@jax.jit
def _aggregate(msg, index, t):
  mesh = plsc.VectorSubcoreMesh(core_axis_name="c", subcore_axis_name="s")
  run = functools.partial(
      pl.kernel,
      out_type=jax.ShapeDtypeStruct((DIM, D), jnp.float32),
      mesh=mesh,
      scratch_types=[
          pltpu.VMEM((NT * SEG_W,), jnp.int32),
          pltpu.VMEM((NT * SEG_W,), jnp.float32),
          pltpu.VMEM((SPAD,), jnp.float32),
          pltpu.VMEM((SPAD,), jnp.int32),
          pltpu.VMEM((SPAD,), jnp.float32),
          pltpu.VMEM((SPAD,), jnp.int32),
          pltpu.VMEM((SEG_W,), jnp.int32),
          pltpu.VMEM((ROWS_CHUNK, D), jnp.float32),
          pltpu.VMEM((ROWS_CHUNK, D), jnp.float32),
          pltpu.SemaphoreType.DMA,
          pltpu.SemaphoreType.DMA,
          pltpu.VMEM_SHARED((NS * NT * SEG_W,), jnp.float32),
          pltpu.VMEM_SHARED((NS * NT * SEG_W,), jnp.int32),
      ],
      compiler_params=pltpu.CompilerParams(needs_layout_passes=False),
  )(_body)
  return run(index, t, msg)


def kernel(msg, index, t, dim_size):
  del dim_size
  return _aggregate(msg, index, t)
